# Initial kernel scaffold; baseline (speedup 1.0000x reference)
import jax
import jax.numpy as jnp
from jax.experimental import pallas as pl


def kernel(x, fc_weight, bias, ffm_tables):
    raise NotImplementedError("write your pallas kernel here")


# speedup vs baseline: 3.6417x; 3.6417x over previous
import functools

import jax
import jax.numpy as jnp
import numpy as np
from jax import lax
from jax.experimental import pallas as pl
from jax.experimental.pallas import tpu as pltpu
from jax.experimental.pallas import tpu_sc as plsc

F = 26
V = 3846
S = 99996
D = 16
B = 4096
NT = 32
RPT = B // NT
G = 32
IDXW = F * G


def _sc_ffm(tab2d, fcpad, xoff1d):
    mesh = plsc.VectorSubcoreMesh(core_axis_name="c", subcore_axis_name="s")

    @functools.partial(
        pl.kernel,
        out_type=jax.ShapeDtypeStruct((B * D,), jnp.float32),
        mesh=mesh,
        scratch_types=[
            pltpu.VMEM((RPT * G,), jnp.int32),
            pltpu.VMEM((IDXW,), jnp.int32),
            pltpu.VMEM((IDXW, D), jnp.float32),
            pltpu.VMEM((G, D), jnp.float32),
            pltpu.VMEM((RPT * D,), jnp.float32),
            pltpu.SemaphoreType.DMA,
        ],
        compiler_params=pltpu.CompilerParams(use_tc_tiling_on_sc=False),
    )
    def kern(tab_hbm, fc_hbm, xo1_hbm, z_hbm,
             xoff_v, idx_v, gbuf, fcbuf, zloc, sem):
        wid = lax.axis_index("s") * 2 + lax.axis_index("c")
        base = wid * RPT
        pltpu.sync_copy(xo1_hbm.at[pl.ds(base * G, RPT * G)], xoff_v)

        @pl.loop(0, RPT)
        def _(r):
            xv0 = xoff_v[pl.ds(r * G, D)]
            xv1 = xoff_v[pl.ds(r * G + D, D)]
            xc0 = jnp.minimum(xv0, S - 1)
            xc1 = jnp.minimum(xv1, S - 1)
            for t in range(F):
                idx_v[pl.ds(t * G, D)] = xc0 + t * S
                idx_v[pl.ds(t * G + D, D)] = xc1 + t * S
            copies = []
            for c in range(IDXW // 128):
                sl = pl.ds(c * 128, 128)
                copies.append(
                    pltpu.async_copy(tab_hbm.at[idx_v.at[sl]], gbuf.at[sl], sem))
            rem = IDXW % 128
            if rem:
                sl = pl.ds(IDXW - rem, rem)
                copies.append(
                    pltpu.async_copy(tab_hbm.at[idx_v.at[sl]], gbuf.at[sl], sem))
            copies.append(
                pltpu.async_copy(
                    fc_hbm.at[xoff_v.at[pl.ds(r * G, G)]], fcbuf, sem))
            for cp in copies:
                cp.wait()

            acc = fcbuf[0]
            for f in range(1, F):
                acc = acc + fcbuf[f]
            for i in range(F - 1):
                for j in range(i + 1, F):
                    acc = acc + gbuf[i * G + j] * gbuf[j * G + i]
            zloc[pl.ds(r * D, D)] = acc

        pltpu.sync_copy(zloc, z_hbm.at[pl.ds(base * D, RPT * D)])

    return kern(tab2d, fcpad, xoff1d)


def _tc_finish(z2d, bias):
    def body(z_ref, b_ref, o_ref):
        s = jnp.sum(z_ref[...], axis=1) + b_ref[0]
        o_ref[...] = jax.nn.sigmoid(s)

    return pl.pallas_call(
        body,
        out_shape=jax.ShapeDtypeStruct((B,), jnp.float32),
    )(z2d, bias)


@jax.jit
def kernel(x, fc_weight, bias, ffm_tables):
    offsets = np.arange(F, dtype=np.int32) * V
    x_off = x.astype(jnp.int32) + jnp.asarray(offsets)[None, :]
    xoff2d = jnp.concatenate(
        [x_off, jnp.full((B, G - F), S, jnp.int32)], axis=1)
    xoff1d = xoff2d.reshape(B * G)

    tab2d = ffm_tables.reshape(F * S, D)
    fcpad = jnp.zeros((S + 1, D), jnp.float32).at[:S, 0].set(fc_weight[:, 0])

    z = _sc_ffm(tab2d, fcpad, xoff1d).reshape(B, D)
    return _tc_finish(z, bias)


# speedup vs baseline: 18.8884x; 5.1867x over previous
import functools

import jax
import jax.numpy as jnp
import numpy as np
from jax import lax
from jax.experimental import pallas as pl
from jax.experimental.pallas import tpu as pltpu
from jax.experimental.pallas import tpu_sc as plsc

F = 26
V = 3846
S = 99996
D = 16
B = 4096
NT = 32
RPT = B // NT
G = 32
NP = 4
FC = 26


def _sc_ffm(t0, t1, t2, t3, xoffT):
    mesh = plsc.VectorSubcoreMesh(core_axis_name="c", subcore_axis_name="s")

    @functools.partial(
        pl.kernel,
        out_type=jax.ShapeDtypeStruct((B * D,), jnp.float32),
        mesh=mesh,
        scratch_types=[
            pltpu.VMEM((G, RPT), jnp.int32),
            pltpu.VMEM((G,), jnp.int32),
            pltpu.VMEM((NP, G, 128), jnp.float32),
            pltpu.VMEM((RPT * D,), jnp.float32),
            pltpu.SemaphoreType.DMA,
        ],
        compiler_params=pltpu.CompilerParams(
            use_tc_tiling_on_sc=False, needs_layout_passes=False),
    )
    def kern(t0_hbm, t1_hbm, t2_hbm, t3_hbm, xo_hbm, z_hbm,
             xoff_v, idx_v, gbuf, zloc, sem):
        wid = lax.axis_index("s") * 2 + lax.axis_index("c")
        base = wid * RPT
        pltpu.sync_copy(xo_hbm.at[:, pl.ds(base, RPT)], xoff_v)
        tabs = (t0_hbm, t1_hbm, t2_hbm, t3_hbm)

        lanes = lax.iota(jnp.int32, 16)

        @pl.loop(0, RPT)
        def _(r):
            rv = jnp.full((16,), r, jnp.int32)
            xv0 = plsc.load_gather(xoff_v, [lanes, rv])
            xv1 = plsc.load_gather(xoff_v, [lanes + D, rv])
            idx_v[pl.ds(0, D)] = xv0
            idx_v[pl.ds(D, D)] = jnp.minimum(xv1, S - 1)
            copies = [
                pltpu.async_copy(tabs[p].at[idx_v], gbuf.at[p], sem)
                for p in range(NP)
            ]
            for cp in copies:
                cp.wait()

            acc = gbuf[FC // 8, 0, pl.ds((FC % 8) * D, D)]
            for f in range(1, F):
                acc = acc + gbuf[FC // 8, f, pl.ds((FC % 8) * D, D)]
            for i in range(F - 1):
                for j in range(i + 1, F):
                    a = gbuf[j // 8, i, pl.ds((j % 8) * D, D)]
                    b = gbuf[i // 8, j, pl.ds((i % 8) * D, D)]
                    acc = acc + a * b
            zloc[pl.ds(r * D, D)] = acc

        pltpu.sync_copy(zloc, z_hbm.at[pl.ds(base * D, RPT * D)])

    return kern(t0, t1, t2, t3, xoffT)


def _tc_finish(z2d, bias):
    def body(z_ref, b_ref, o_ref):
        o_ref[...] = jax.nn.sigmoid(jnp.sum(z_ref[...], axis=1) + b_ref[0])

    return pl.pallas_call(
        body,
        out_shape=jax.ShapeDtypeStruct((B,), jnp.float32),
    )(z2d, bias)


@jax.jit
def kernel(x, fc_weight, bias, ffm_tables):
    offsets = np.arange(F, dtype=np.int32) * V
    x_off = x.astype(jnp.int32) + jnp.asarray(offsets)[None, :]
    xoffT = jnp.concatenate(
        [x_off.T, jnp.full((G - F, B), S, jnp.int32)], axis=0)

    packs = []
    for i in range(3):
        packs.append(
            ffm_tables[8 * i:8 * i + 8].transpose(1, 0, 2).reshape(S, 128))
    fc16 = jnp.concatenate([fc_weight, jnp.zeros((S, D - 1), jnp.float32)], 1)
    last = jnp.concatenate([ffm_tables[24:26], fc16[None]], axis=0)
    t3 = jnp.pad(last.transpose(1, 0, 2).reshape(S, 48), ((0, 0), (0, 80)))
    packs.append(t3)

    z = _sc_ffm(*packs, xoffT)
    return _tc_finish(z.reshape(B, D), bias)
